# Initial kernel scaffold; baseline (speedup 1.0000x reference)
import jax
import jax.numpy as jnp
from jax.experimental import pallas as pl


def kernel(x, W1, b1, g1, bt1, W2, b2, g2, bt2, W3, b3, g3, bt3):
    raise NotImplementedError("write your pallas kernel here")


# speedup vs baseline: 10.8264x; 10.8264x over previous
import functools

import jax
import jax.numpy as jnp
from jax import lax
from jax.experimental import pallas as pl
from jax.experimental.pallas import tpu as pltpu
from jax.experimental.pallas import tpu_sc as plsc

B, N, D = 2, 4096, 16
EMB, OUT, K = 64, 64, 16
E = B * N * K

RB = 256
MB = 1024


def _knn_body(xc_ref, yc_ref, xr_ref, yr_ref, out_ref):
    b = pl.program_id(0)
    xc = xc_ref[0]
    yc = yc_ref[0]
    xr = xr_ref[0]
    yr = yr_ref[0]
    sqc = xc * xc + yc * yc
    sqr = xr * xr + yr * yr
    bf = lambda v: v.astype(jnp.bfloat16).astype(jnp.float32)
    e = bf(xc) * bf(xr) + bf(yc) * bf(yr)
    d2 = (sqc + sqr) - 2.0 * e
    iota = lax.broadcasted_iota(jnp.int32, (RB, N), 1)
    cols = []
    for _ in range(K):
        m = jnp.min(d2, axis=1, keepdims=True)
        pos = jnp.min(jnp.where(d2 == m, iota, N), axis=1, keepdims=True)
        cols.append(pos)
        d2 = jnp.where(iota == pos, jnp.float32(jnp.inf), d2)
    out_ref[0] = jnp.concatenate(cols, axis=1) + b * N


def _knn_idx(x):
    xc = x[:, :, 8].reshape(B, N, 1)
    yc = x[:, :, 9].reshape(B, N, 1)
    xr = x[:, :, 8].reshape(B, 1, N)
    yr = x[:, :, 9].reshape(B, 1, N)
    return pl.pallas_call(
        _knn_body,
        grid=(B, N // RB),
        in_specs=[
            pl.BlockSpec((1, RB, 1), lambda b, i: (b, i, 0)),
            pl.BlockSpec((1, RB, 1), lambda b, i: (b, i, 0)),
            pl.BlockSpec((1, 1, N), lambda b, i: (b, 0, 0)),
            pl.BlockSpec((1, 1, N), lambda b, i: (b, 0, 0)),
        ],
        out_specs=pl.BlockSpec((1, RB, K), lambda b, i: (b, i, 0)),
        out_shape=jax.ShapeDtypeStruct((B, N, K), jnp.int32),
    )(xc, yc, xr, yr)


def _sc_gather(table, idx2d):
    info = plsc.get_sparse_core_info()
    nw = info.num_cores * info.num_subcores
    epw = E // nw
    cpw = epw // 128
    grp = 8
    mesh = plsc.VectorSubcoreMesh(core_axis_name="c", subcore_axis_name="s")

    @functools.partial(
        pl.kernel,
        mesh=mesh,
        compiler_params=pltpu.CompilerParams(use_tc_tiling_on_sc=False),
        out_type=jax.ShapeDtypeStruct((E, D), jnp.float32),
        scratch_types=[
            pltpu.VMEM((cpw, 128), jnp.int32),
            pltpu.VMEM((epw, D), jnp.float32),
            pltpu.SemaphoreType.DMA,
        ],
    )
    def gather(table_hbm, idx_hbm, out_hbm, idx_v, rows_v, sem):
        wid = lax.axis_index("s") * info.num_cores + lax.axis_index("c")
        pltpu.sync_copy(idx_hbm.at[pl.ds(wid * cpw, cpw)], idx_v)

        def group(g, carry):
            copies = []
            for j in range(grp):
                c = g * grp + j
                copies.append(pltpu.async_copy(
                    table_hbm.at[idx_v.at[c]],
                    rows_v.at[pl.ds(c * 128, 128)],
                    sem,
                ))
            for cp in copies:
                cp.wait()
            return carry

        lax.fori_loop(0, cpw // grp, group, 0)
        pltpu.sync_copy(rows_v, out_hbm.at[pl.ds(wid * epw, epw)])

    return gather(table, idx2d)


def _mlp_body(x_ref, nbr_ref,
              w1_ref, b1_ref, g1_ref, t1_ref,
              w2_ref, b2_ref, g2_ref, t2_ref,
              w3_ref, b3_ref, g3_ref, t3_ref,
              out_ref):
    cen = x_ref[...]
    nbr = nbr_ref[...]
    cen_rep = jnp.broadcast_to(cen[:, None, :], (MB, K, D)).reshape(MB * K, D)
    h = jnp.concatenate([cen_rep, nbr - cen_rep], axis=1)
    for w_ref, b_ref, g_ref, t_ref in (
        (w1_ref, b1_ref, g1_ref, t1_ref),
        (w2_ref, b2_ref, g2_ref, t2_ref),
        (w3_ref, b3_ref, g3_ref, t3_ref),
    ):
        h = jnp.dot(h, w_ref[...], preferred_element_type=jnp.float32)
        h = h + b_ref[...]
        mu = jnp.mean(h, axis=1, keepdims=True)
        var = jnp.mean((h - mu) ** 2, axis=1, keepdims=True)
        h = (h - mu) / jnp.sqrt(var + 1e-5) * g_ref[...] + t_ref[...]
        h = h * 0.5 * (1.0 + lax.erf(h * jnp.float32(0.7071067811865476)))
    out_ref[...] = jnp.mean(h.reshape(MB, K, OUT), axis=1)


def _mlp(xf, nbr, params):
    full = lambda shape: pl.BlockSpec(shape, lambda i: tuple(0 for _ in shape))
    in_specs = [
        pl.BlockSpec((MB, D), lambda i: (i, 0)),
        pl.BlockSpec((MB * K, D), lambda i: (i, 0)),
    ]
    args = [xf, nbr]
    for w, b, g, t in params:
        fin = w.shape[0]
        fout = w.shape[1]
        in_specs += [full((fin, fout)), full((1, fout)),
                     full((1, fout)), full((1, fout))]
        args += [w, b.reshape(1, fout), g.reshape(1, fout),
                 t.reshape(1, fout)]
    return pl.pallas_call(
        _mlp_body,
        grid=(B * N // MB,),
        in_specs=in_specs,
        out_specs=pl.BlockSpec((MB, OUT), lambda i: (i, 0)),
        out_shape=jax.ShapeDtypeStruct((B * N, OUT), jnp.float32),
    )(*args)


def kernel(x, W1, b1, g1, bt1, W2, b2, g2, bt2, W3, b3, g3, bt3):
    gidx = _knn_idx(x)
    table = x.reshape(B * N, D)
    nbr = _sc_gather(table, gidx.reshape(E // 128, 128))
    out = _mlp(table, nbr,
               ((W1, b1, g1, bt1), (W2, b2, g2, bt2), (W3, b3, g3, bt3)))
    return out.reshape(B, N, OUT)
